# baseline (speedup 1.0000x reference)
import functools
import math

import jax
import jax.numpy as jnp
from jax.experimental import pallas as pl
from jax.experimental.pallas import tpu as pltpu

_HIGH = jax.lax.Precision.HIGHEST


def _body(H, kk, scale, x_ref, wr_ref, br_ref, wo_ref, bout_ref, s_ref,
          o_ref, bias_scr):
    h = pl.program_id(1)

    @pl.when(h == 0)
    def _():
        bias_scr[...] = jnp.log(s_ref[0] + 1e-8)

    xb = x_ref[0]
    wq = wr_ref[pl.ds(h, 1)][0]
    wk = wr_ref[pl.ds(H + h, 1)][0]
    wv = wr_ref[pl.ds(2 * H + h, 1)][0]
    bq = br_ref[pl.ds(h, 1)][0]
    bk = br_ref[pl.ds(H + h, 1)][0]
    bv = br_ref[pl.ds(2 * H + h, 1)][0]

    dn_t = (((1,), (1,)), ((), ()))
    q = jax.lax.dot_general(xb, wq, dn_t, precision=jax.lax.Precision.DEFAULT) + bq[None, :]
    k = jax.lax.dot_general(xb, wk, dn_t, precision=jax.lax.Precision.DEFAULT) + bk[None, :]
    v = jax.lax.dot_general(xb, wv, dn_t, precision=jax.lax.Precision.DEFAULT) + bv[None, :]

    scores = jax.lax.dot_general(k, q, dn_t, precision=jax.lax.Precision.DEFAULT) * scale
    scores = scores + bias_scr[...]

    m = jnp.max(scores, axis=0, keepdims=True)
    rmin = jnp.min(scores, axis=0, keepdims=True)
    top = jnp.uint32(0x80000000)
    u = jax.lax.bitcast_convert_type(scores, jnp.uint32)
    keys = jnp.where(u >= top, ~u, u | top)
    u_hi = jax.lax.bitcast_convert_type(m, jnp.uint32)
    u_lo = jax.lax.bitcast_convert_type(rmin, jnp.uint32)
    hi0 = jnp.where(u_hi >= top, ~u_hi, u_hi | top)
    lo0 = jnp.where(u_lo >= top, ~u_lo, u_lo | top)

    def step(_, lh):
        lo, hi = lh
        mid = lo + ((hi - lo + jnp.uint32(1)) >> 1)
        cnt = jnp.sum((keys >= mid).astype(jnp.int32), axis=0,
                      keepdims=True)
        ge = cnt >= kk
        return jnp.where(ge, mid, lo), jnp.where(ge, hi, mid - jnp.uint32(1))

    lo, _ = jax.lax.fori_loop(0, 32, step, (lo0, hi0))

    sel = keys >= lo
    p = jnp.where(sel, jnp.exp(scores - m), 0.0)

    vx = jnp.concatenate([v, jnp.ones((v.shape[0], 1), jnp.float32)],
                         axis=1)
    dn_n = (((1,), (0,)), ((), ()))
    dn_c0 = (((0,), (0,)), ((), ()))
    hd = v.shape[1]
    o_ext = jax.lax.dot_general(p, vx, dn_c0,
                                precision=jax.lax.Precision.DEFAULT)
    o = o_ext[:, :hd] * (1.0 / o_ext[:, hd:hd + 1])
    proj = jax.lax.dot_general(o, wo_ref[pl.ds(h, 1)][0], dn_n,
                               precision=jax.lax.Precision.DEFAULT)

    @pl.when(h == 0)
    def _():
        o_ref[0] = proj + bout_ref[0][None, :]

    @pl.when(h != 0)
    def _():
        o_ref[0] = o_ref[0] + proj


def kernel(x, Wqkv, bqkv, Wout, bout, S_struc):
    L, B, D = x.shape
    H = S_struc.shape[0]
    hd = D // H
    kk = max(1, int(0.1 * L))
    scale = 1.0 / math.sqrt(hd)

    Wr = Wqkv.reshape(3 * H, hd, D)
    br = bqkv.reshape(3 * H, hd)
    Wo = jnp.transpose(Wout.reshape(D, H, hd), (1, 2, 0))
    bo = bout.reshape(1, D)

    body = functools.partial(_body, H, kk, scale)
    xt = jnp.transpose(x, (1, 0, 2))

    out = pl.pallas_call(
        body,
        grid=(B, H),
        in_specs=[
            pl.BlockSpec((1, L, D), lambda b, h: (b, 0, 0)),
            pl.BlockSpec((3 * H, hd, D), lambda b, h: (0, 0, 0)),
            pl.BlockSpec((3 * H, hd), lambda b, h: (0, 0)),
            pl.BlockSpec((H, hd, D), lambda b, h: (0, 0, 0)),
            pl.BlockSpec((1, D), lambda b, h: (0, 0)),
            pl.BlockSpec((1, L, L), lambda b, h: (b, 0, 0)),
        ],
        out_specs=pl.BlockSpec((1, L, D), lambda b, h: (b, 0, 0)),
        out_shape=jax.ShapeDtypeStruct((B, L, D), jnp.float32),
        scratch_shapes=[pltpu.VMEM((L, L), jnp.float32)],
        compiler_params=pltpu.CompilerParams(
            dimension_semantics=("arbitrary", "arbitrary")),
    )(xt, Wr, br, Wo, bo, jnp.transpose(S_struc, (0, 2, 1)))
    return jnp.transpose(out, (1, 0, 2))


# speedup vs baseline: 1.3194x; 1.3194x over previous
import functools
import math

import jax
import jax.numpy as jnp
from jax.experimental import pallas as pl
from jax.experimental.pallas import tpu as pltpu

_DEF = jax.lax.Precision.DEFAULT


def _body(H, kk, scale, x_ref, wr_ref, br_ref, wo_ref, bout_ref, s_ref,
          o_ref, bias_scr):
    h2 = pl.program_id(1)
    ha = 2 * h2
    hb = 2 * h2 + 1

    @pl.when(h2 == 0)
    def _():
        bias_scr[...] = jnp.log(s_ref[0] + 1e-8)

    xb = x_ref[0]
    dn_t = (((1,), (1,)), ((), ()))

    def qkv(hh):
        wq = wr_ref[pl.ds(hh, 1)][0]
        wk = wr_ref[pl.ds(H + hh, 1)][0]
        wv = wr_ref[pl.ds(2 * H + hh, 1)][0]
        bq = br_ref[pl.ds(hh, 1)][0]
        bk = br_ref[pl.ds(H + hh, 1)][0]
        bv = br_ref[pl.ds(2 * H + hh, 1)][0]
        q = jax.lax.dot_general(xb, wq, dn_t, precision=_DEF) + bq[None, :]
        k = jax.lax.dot_general(xb, wk, dn_t, precision=_DEF) + bk[None, :]
        v = jax.lax.dot_general(xb, wv, dn_t, precision=_DEF) + bv[None, :]
        s = jax.lax.dot_general(k, q, dn_t, precision=_DEF) * scale
        return s + bias_scr[...], v

    sA, vA = qkv(ha)
    sB, vB = qkv(hb)

    mA = jnp.max(sA, axis=0, keepdims=True)
    mB = jnp.max(sB, axis=0, keepdims=True)
    loA0 = jnp.min(sA, axis=0, keepdims=True)
    loB0 = jnp.min(sB, axis=0, keepdims=True)

    def count(s, t):
        return jnp.sum((s >= t).astype(jnp.float32), axis=0, keepdims=True)

    def step(_, state):
        loA, hiA, loB, hiB = state
        midA = 0.5 * (loA + hiA)
        midB = 0.5 * (loB + hiB)
        geA = count(sA, midA) >= kk
        geB = count(sB, midB) >= kk
        return (jnp.where(geA, midA, loA), jnp.where(geA, hiA, midA),
                jnp.where(geB, midB, loB), jnp.where(geB, hiB, midB))

    loA, hiA, loB, hiB = jax.lax.fori_loop(
        0, 13, step, (loA0, mA, loB0, mB))

    cA = count(sA, hiA).astype(jnp.int32)
    cB = count(sB, hiB).astype(jnp.int32)
    rA0 = kk - cA
    rB0 = kk - cB
    dA0 = (rA0 <= 0).astype(jnp.int32)
    dB0 = (rB0 <= 0).astype(jnp.int32)
    tA0 = jnp.where(dA0 == 1, hiA, loA)
    tB0 = jnp.where(dB0 == 1, hiB, loB)

    def ext_cond(state):
        dA, dB = state[0], state[4]
        return jnp.minimum(jnp.min(dA), jnp.min(dB)) == 0

    def ext_half(s, lo, done, r, thr, ub):
        cand = (s >= lo) & (s < ub)
        mc = jnp.max(jnp.where(cand, s, -jnp.inf), axis=0, keepdims=True)
        c_m = jnp.sum((s == mc).astype(jnp.int32), axis=0, keepdims=True)
        active = done == 0
        take = active & (r <= c_m)
        thr = jnp.where(take, mc, thr)
        done = jnp.where(take, 1, done)
        cont = active & jnp.logical_not(take)
        r = jnp.where(cont, r - c_m, r)
        ub = jnp.where(cont, mc, ub)
        return done, r, thr, ub

    def ext_body(state):
        dA, rA, tA, uA, dB, rB, tB, uB = state
        dA, rA, tA, uA = ext_half(sA, loA, dA, rA, tA, uA)
        dB, rB, tB, uB = ext_half(sB, loB, dB, rB, tB, uB)
        return dA, rA, tA, uA, dB, rB, tB, uB

    st = jax.lax.while_loop(
        ext_cond, ext_body,
        (dA0, rA0, tA0, hiA, dB0, rB0, tB0, hiB))
    thrA, thrB = st[2], st[6]

    pA = jnp.where(sA >= thrA, jnp.exp(sA - mA), 0.0)
    pB = jnp.where(sB >= thrB, jnp.exp(sB - mB), 0.0)

    dn_n = (((1,), (0,)), ((), ()))
    dn_c0 = (((0,), (0,)), ((), ()))
    hd = vA.shape[1]
    ones_col = jnp.ones((vA.shape[0], 1), jnp.float32)

    def out_half(p, v, hh):
        vx = jnp.concatenate([v, ones_col], axis=1)
        o_ext = jax.lax.dot_general(p, vx, dn_c0,
                                    precision=_DEF)
        o = o_ext[:, :hd] * (1.0 / o_ext[:, hd:hd + 1])
        return jax.lax.dot_general(o, wo_ref[pl.ds(hh, 1)][0], dn_n,
                                   precision=_DEF)

    proj = out_half(pA, vA, ha) + out_half(pB, vB, hb)

    @pl.when(h2 == 0)
    def _():
        o_ref[0] = proj + bout_ref[0][None, :]

    @pl.when(h2 != 0)
    def _():
        o_ref[0] = o_ref[0] + proj


def kernel(x, Wqkv, bqkv, Wout, bout, S_struc):
    L, B, D = x.shape
    H = S_struc.shape[0]
    hd = D // H
    kk = max(1, int(0.1 * L))
    scale = 1.0 / math.sqrt(hd)

    Wr = Wqkv.reshape(3 * H, hd, D)
    br = bqkv.reshape(3 * H, hd)
    Wo = jnp.transpose(Wout.reshape(D, H, hd), (1, 2, 0))
    bo = bout.reshape(1, D)

    body = functools.partial(_body, H, kk, scale)
    xt = jnp.transpose(x, (1, 0, 2))

    out = pl.pallas_call(
        body,
        grid=(B, H // 2),
        in_specs=[
            pl.BlockSpec((1, L, D), lambda b, h: (b, 0, 0)),
            pl.BlockSpec((3 * H, hd, D), lambda b, h: (0, 0, 0)),
            pl.BlockSpec((3 * H, hd), lambda b, h: (0, 0)),
            pl.BlockSpec((H, hd, D), lambda b, h: (0, 0, 0)),
            pl.BlockSpec((1, D), lambda b, h: (0, 0)),
            pl.BlockSpec((1, L, L), lambda b, h: (b, 0, 0)),
        ],
        out_specs=pl.BlockSpec((1, L, D), lambda b, h: (b, 0, 0)),
        out_shape=jax.ShapeDtypeStruct((B, L, D), jnp.float32),
        scratch_shapes=[pltpu.VMEM((L, L), jnp.float32)],
        compiler_params=pltpu.CompilerParams(
            dimension_semantics=("arbitrary", "arbitrary")),
    )(xt, Wr, br, Wo, bo, jnp.transpose(S_struc, (0, 2, 1)))
    return jnp.transpose(out, (1, 0, 2))
